# baseline (speedup 1.0000x reference)
import jax
import jax.numpy as jnp
from jax import lax
from jax.experimental import pallas as pl
from jax.experimental.pallas import tpu as pltpu
from jax.experimental.pallas import tpu_sc as plsc

B, S, D = 4, 8192, 2048
S_TC = 4608
S_SC = S - S_TC

L = 16
NC, NS = 2, 16
NW = NC * NS
COLS = (B * D) // NW
DW = COLS // L
WPB = D // COLS
CH = 64
NCH = S_SC // CH
NB = 4


def _sc_body(x_hbm, val_hbm, idx_hbm, buf0, buf1, buf2, buf3, valbuf, idxbuf,
             sem0, sem1, sem2, sem3):
    bufs = (buf0, buf1, buf2, buf3)
    sems = (sem0, sem1, sem2, sem3)

    wid = lax.axis_index("s") * NC + lax.axis_index("c")
    b = wid // WPB
    d0 = (wid % WPB) * COLS

    def src(c):
        return x_hbm.at[b, pl.ds(S_TC + c * CH, CH), pl.ds(d0, COLS)]

    for k in range(NB):
        pltpu.async_copy(src(k), bufs[k], sems[k])

    def scan_chunk(buf, base, carry):
        def s_body(s, carry):
            vals, idxs = carry
            svec = jnp.full((L,), S_TC + base + s, dtype=jnp.int32)
            nv, ni = [], []
            for g in range(DW):
                v = buf[s, pl.ds(g * L, L)]
                m = v > vals[g]
                nv.append(jnp.where(m, v, vals[g]))
                ni.append(jnp.where(m, svec, idxs[g]))
            return (tuple(nv), tuple(ni))

        return lax.fori_loop(0, CH, s_body, carry)

    def step(c, bi, carry):
        pltpu.make_async_copy(src(c), bufs[bi], sems[bi]).wait()
        carry = scan_chunk(bufs[bi], c * CH, carry)

        @pl.when(c + NB < NCH)
        def _():
            pltpu.async_copy(src(c + NB), bufs[bi], sems[bi])

        return carry

    neg = jnp.full((L,), -jnp.inf, dtype=jnp.float32)
    zero = jnp.zeros((L,), dtype=jnp.int32)
    carry = (tuple(neg for _ in range(DW)), tuple(zero for _ in range(DW)))

    def ring_body(p, carry):
        c0 = NB * p
        for k in range(NB):
            carry = step(c0 + k, k, carry)
        return carry

    carry = lax.fori_loop(0, NCH // NB, ring_body, carry)
    for c in range(NB * (NCH // NB), NCH):
        carry = step(c, c % NB, carry)

    vals, idxs = carry
    for g in range(DW):
        valbuf[pl.ds(g * L, L)] = vals[g]
        idxbuf[pl.ds(g * L, L)] = idxs[g]
    pltpu.sync_copy(valbuf, val_hbm.at[b, pl.ds(d0, COLS)])
    pltpu.sync_copy(idxbuf, idx_hbm.at[b, pl.ds(d0, COLS)])


def _sc_argmax(x):
    mesh = plsc.VectorSubcoreMesh(
        core_axis_name="c", subcore_axis_name="s",
        num_cores=NC, num_subcores=NS,
    )
    f = pl.kernel(
        _sc_body,
        out_type=(
            jax.ShapeDtypeStruct((B, D), jnp.float32),
            jax.ShapeDtypeStruct((B, D), jnp.int32),
        ),
        mesh=mesh,
        scratch_types=[
            pltpu.VMEM((CH, COLS), jnp.float32),
            pltpu.VMEM((CH, COLS), jnp.float32),
            pltpu.VMEM((CH, COLS), jnp.float32),
            pltpu.VMEM((CH, COLS), jnp.float32),
            pltpu.VMEM((COLS,), jnp.float32),
            pltpu.VMEM((COLS,), jnp.int32),
            pltpu.SemaphoreType.DMA,
            pltpu.SemaphoreType.DMA,
            pltpu.SemaphoreType.DMA,
            pltpu.SemaphoreType.DMA,
        ],
    )
    return f(x)


D_BLK = 512
S_BLK = 512
N_SB = S_TC // S_BLK


def _tc_body(x_ref, v_ref, i_ref, acc_v, acc_i):
    s = pl.program_id(1)
    vals = x_ref[...]
    lm = jnp.max(vals, axis=1)
    iota = lax.broadcasted_iota(jnp.int32, vals.shape, 1)
    li = jnp.min(jnp.where(vals == lm[:, None, :], iota, S), axis=1)
    li = li + s * S_BLK

    @pl.when(s == 0)
    def _():
        acc_v[...] = lm
        acc_i[...] = li

    @pl.when(s > 0)
    def _():
        m = lm > acc_v[...]
        acc_v[...] = jnp.where(m, lm, acc_v[...])
        acc_i[...] = jnp.where(m, li, acc_i[...])

    @pl.when(s == N_SB - 1)
    def _():
        v_ref[...] = acc_v[...]
        i_ref[...] = acc_i[...]


def _tc_argmax(x):
    return pl.pallas_call(
        _tc_body,
        grid=(D // D_BLK, N_SB),
        in_specs=[pl.BlockSpec((B, S_BLK, D_BLK), lambda d, s: (0, s, d))],
        out_specs=(
            pl.BlockSpec((B, D_BLK), lambda d, s: (0, d)),
            pl.BlockSpec((B, D_BLK), lambda d, s: (0, d)),
        ),
        out_shape=(
            jax.ShapeDtypeStruct((B, D), jnp.float32),
            jax.ShapeDtypeStruct((B, D), jnp.int32),
        ),
        scratch_shapes=[
            pltpu.VMEM((B, D_BLK), jnp.float32),
            pltpu.VMEM((B, D_BLK), jnp.int32),
        ],
    )(x)


def _merge_body(tcv_ref, tci_ref, scv_ref, sci_ref, o_ref):
    m = tcv_ref[...] >= scv_ref[...]
    o_ref[...] = jnp.where(m, tci_ref[...], sci_ref[...])


def _merge(tcv, tci, scv, sci):
    return pl.pallas_call(
        _merge_body,
        out_shape=jax.ShapeDtypeStruct((B, D), jnp.int32),
    )(tcv, tci, scv, sci)


def kernel(x):
    scv, sci = _sc_argmax(x)
    tcv, tci = _tc_argmax(x)
    return _merge(tcv, tci, scv, sci)


# speedup vs baseline: 1.1970x; 1.1970x over previous
import jax
import jax.numpy as jnp
from jax import lax
from jax.experimental import pallas as pl
from jax.experimental.pallas import tpu as pltpu

B, S, D = 4, 8192, 2048


def kernel(x):
    S_BLK = 256
    n_s = S // S_BLK

    def body(x_ref, o_ref, acc_v, acc_i):
        s = pl.program_id(0)
        vals = x_ref[...]
        lm = jnp.max(vals, axis=1)
        iota = lax.broadcasted_iota(jnp.int32, vals.shape, 1)
        li = jnp.min(jnp.where(vals == lm[:, None, :], iota, S), axis=1)
        li = li + s * S_BLK

        @pl.when(s == 0)
        def _():
            acc_v[...] = lm
            acc_i[...] = li

        @pl.when(s > 0)
        def _():
            m = lm > acc_v[...]
            acc_v[...] = jnp.where(m, lm, acc_v[...])
            acc_i[...] = jnp.where(m, li, acc_i[...])

        @pl.when(s == n_s - 1)
        def _():
            o_ref[...] = acc_i[...]

    return pl.pallas_call(
        body,
        grid=(n_s,),
        in_specs=[pl.BlockSpec((B, S_BLK, D), lambda s: (0, s, 0))],
        out_specs=pl.BlockSpec((B, D), lambda s: (0, 0)),
        out_shape=jax.ShapeDtypeStruct((B, D), jnp.int32),
        scratch_shapes=[
            pltpu.VMEM((B, D), jnp.float32),
            pltpu.VMEM((B, D), jnp.int32),
        ],
    )(x)
